# baseline (speedup 1.0000x reference)
import functools
import math

import jax
import jax.numpy as jnp
from jax.experimental import pallas as pl
from jax.experimental.pallas import tpu as pltpu

E = 8
TOP_K = 2
RANK = 8
ER = E * RANK

_INV_SQRT2 = 1.0 / math.sqrt(2.0)


def _moe_body(x_ref, ew_ref, ei_ref, w1a_ref, w1b_ref, w2a_ref, w2b_ref,
              y_ref, *, block_tokens):
    xb = x_ref[...]
    a1 = jax.lax.dot_general(xb, w1a_ref[...], (((1,), (1,)), ((), ())),
                             preferred_element_type=jnp.float32)
    slot_expert = jax.lax.broadcasted_iota(jnp.int32, (block_tokens, ER), 1) // RANK
    acc = jnp.zeros((block_tokens, ER), jnp.float32)
    for k in range(TOP_K):
        e_k = ei_ref[:, k].reshape(block_tokens, 1)
        mask = (slot_expert == e_k).astype(jnp.float32)
        ak = (a1 * mask).astype(jnp.bfloat16)
        t = jax.lax.dot_general(ak, w1b_ref[...], (((1,), (0,)), ((), ())),
                                preferred_element_type=jnp.float32)
        g = t * 0.5 * (1.0 + jax.lax.erf(t * _INV_SQRT2))
        s = jax.lax.dot_general(g.astype(jnp.bfloat16), w2a_ref[...],
                                (((1,), (1,)), ((), ())),
                                preferred_element_type=jnp.float32)
        w_k = ew_ref[:, k].reshape(block_tokens, 1)
        acc = acc + s * mask * w_k
    y_ref[...] = jax.lax.dot_general(acc.astype(jnp.bfloat16), w2b_ref[...],
                                     (((1,), (0,)), ((), ())),
                                     preferred_element_type=jnp.float32)


@functools.partial(jax.jit, static_argnames=("block_tokens",))
def _moe(xf, ew, ei, w1_A, w1_B, w2_A, w2_B, block_tokens):
    n, h = xf.shape
    ff = w1_B.shape[1]
    grid = (n // block_tokens,)
    return pl.pallas_call(
        functools.partial(_moe_body, block_tokens=block_tokens),
        grid=grid,
        in_specs=[
            pl.BlockSpec((block_tokens, h), lambda i: (i, 0)),
            pl.BlockSpec((block_tokens, TOP_K), lambda i: (i, 0)),
            pl.BlockSpec((block_tokens, TOP_K), lambda i: (i, 0)),
            pl.BlockSpec((ER, h), lambda i: (0, 0)),
            pl.BlockSpec((ER, ff), lambda i: (0, 0)),
            pl.BlockSpec((ER, ff), lambda i: (0, 0)),
            pl.BlockSpec((ER, h), lambda i: (0, 0)),
        ],
        out_specs=pl.BlockSpec((block_tokens, h), lambda i: (i, 0)),
        out_shape=jax.ShapeDtypeStruct((n, h), jnp.float32),
        compiler_params=pltpu.CompilerParams(
            dimension_semantics=("parallel",)),
    )(xf, ew, ei, w1_A, w1_B, w2_A, w2_B)


def kernel(x, expert_weights, expert_indices, w1_A, w1_B, w2_A, w2_B):
    in_shape = x.shape
    n = in_shape[0] * in_shape[1]
    h = in_shape[2]
    xf = x.reshape(n, h).astype(jnp.bfloat16)
    ew = expert_weights.reshape(n, TOP_K).astype(jnp.float32)
    ei = expert_indices.reshape(n, TOP_K).astype(jnp.int32)
    y = _moe(xf, ew, ei, w1_A.astype(jnp.bfloat16), w1_B.astype(jnp.bfloat16),
             w2_A.astype(jnp.bfloat16), w2_B.astype(jnp.bfloat16),
             block_tokens=256)
    return y.reshape(in_shape)


# speedup vs baseline: 1.0977x; 1.0977x over previous
import functools
import math

import jax
import jax.numpy as jnp
from jax.experimental import pallas as pl
from jax.experimental.pallas import tpu as pltpu

E = 8
TOP_K = 2
RANK = 8
ER = E * RANK

_INV_SQRT2 = 1.0 / math.sqrt(2.0)


def _moe_body(x_ref, ew_ref, ei_ref, w1a_ref, w1b_ref, w2a_ref, w2b_ref,
              y_ref, *, block_tokens):
    xb = x_ref[...]
    a1 = jax.lax.dot_general(xb, w1a_ref[...], (((1,), (1,)), ((), ())),
                             preferred_element_type=jnp.float32)
    slot_expert = jax.lax.broadcasted_iota(jnp.int32, (block_tokens, ER), 1) // RANK
    acc = jnp.zeros((block_tokens, ER), jnp.float32)
    for k in range(TOP_K):
        e_k = ei_ref[:, k].reshape(block_tokens, 1)
        mask = (slot_expert == e_k).astype(jnp.float32)
        ak = a1 * mask
        t = jax.lax.dot_general(ak, w1b_ref[...], (((1,), (0,)), ((), ())),
                                preferred_element_type=jnp.float32)
        g = t * 0.5 * (1.0 + jax.lax.erf(t * _INV_SQRT2))
        s = jax.lax.dot_general(g, w2a_ref[...], (((1,), (1,)), ((), ())),
                                preferred_element_type=jnp.float32)
        w_k = ew_ref[:, k].reshape(block_tokens, 1)
        acc = acc + s * mask * w_k
    y_ref[...] = jax.lax.dot_general(acc, w2b_ref[...], (((1,), (0,)), ((), ())),
                                     preferred_element_type=jnp.float32)


@functools.partial(jax.jit, static_argnames=("block_tokens",))
def _moe(xf, ew, ei, w1_A, w1_B, w2_A, w2_B, block_tokens):
    n, h = xf.shape
    ff = w1_B.shape[1]
    grid = (n // block_tokens,)
    return pl.pallas_call(
        functools.partial(_moe_body, block_tokens=block_tokens),
        grid=grid,
        in_specs=[
            pl.BlockSpec((block_tokens, h), lambda i: (i, 0)),
            pl.BlockSpec((block_tokens, TOP_K), lambda i: (i, 0)),
            pl.BlockSpec((block_tokens, TOP_K), lambda i: (i, 0)),
            pl.BlockSpec((ER, h), lambda i: (0, 0)),
            pl.BlockSpec((ER, ff), lambda i: (0, 0)),
            pl.BlockSpec((ER, ff), lambda i: (0, 0)),
            pl.BlockSpec((ER, h), lambda i: (0, 0)),
        ],
        out_specs=pl.BlockSpec((block_tokens, h), lambda i: (i, 0)),
        out_shape=jax.ShapeDtypeStruct((n, h), jnp.float32),
        compiler_params=pltpu.CompilerParams(
            dimension_semantics=("parallel",)),
    )(xf, ew, ei, w1_A, w1_B, w2_A, w2_B)


def kernel(x, expert_weights, expert_indices, w1_A, w1_B, w2_A, w2_B):
    in_shape = x.shape
    n = in_shape[0] * in_shape[1]
    h = in_shape[2]
    xf = x.reshape(n, h)
    ew = expert_weights.reshape(n, TOP_K).astype(jnp.float32)
    ei = expert_indices.reshape(n, TOP_K).astype(jnp.int32)
    y = _moe(xf, ew, ei, w1_A, w1_B, w2_A, w2_B, block_tokens=256)
    return y.reshape(in_shape)


# speedup vs baseline: 2.0603x; 1.8770x over previous
import functools
import math

import jax
import jax.numpy as jnp
from jax.experimental import pallas as pl
from jax.experimental.pallas import tpu as pltpu

E = 8
TOP_K = 2
RANK = 8
ER = E * RANK

_INV_SQRT2 = 1.0 / math.sqrt(2.0)


def _moe_body(x_ref, ew_ref, ei_ref, w1a_ref, w1b_ref, w2a_ref, w2b_ref,
              y_ref, *, block_tokens):
    xb = x_ref[:, 0, :]
    a1 = jax.lax.dot_general(xb, w1a_ref[...], (((1,), (1,)), ((), ())),
                             preferred_element_type=jnp.float32)
    slot_expert = jax.lax.broadcasted_iota(jnp.int32, (block_tokens, ER), 1) // RANK
    acc = jnp.zeros((block_tokens, ER), jnp.float32)
    for k in range(TOP_K):
        e_k = ei_ref[:, k].reshape(block_tokens, 1)
        mask = (slot_expert == e_k).astype(jnp.float32)
        ak = a1 * mask
        t = jax.lax.dot_general(ak, w1b_ref[...], (((1,), (0,)), ((), ())),
                                preferred_element_type=jnp.float32)
        g = t * 0.5 * (1.0 + jax.lax.erf(t * _INV_SQRT2))
        s = jax.lax.dot_general(g, w2a_ref[...], (((1,), (1,)), ((), ())),
                                preferred_element_type=jnp.float32)
        w_k = ew_ref[:, k].reshape(block_tokens, 1)
        acc = acc + s * mask * w_k
    y_ref[:, 0, :] = jax.lax.dot_general(acc, w2b_ref[...],
                                         (((1,), (0,)), ((), ())),
                                         preferred_element_type=jnp.float32)


@functools.partial(jax.jit, static_argnames=("block_tokens",))
def _moe(x, ew, ei, w1_A, w1_B, w2_A, w2_B, block_tokens):
    n, b, h = x.shape
    ff = w1_B.shape[1]
    grid = (n // block_tokens,)
    return pl.pallas_call(
        functools.partial(_moe_body, block_tokens=block_tokens),
        grid=grid,
        in_specs=[
            pl.BlockSpec((block_tokens, 1, h), lambda i: (i, 0, 0)),
            pl.BlockSpec((block_tokens, TOP_K), lambda i: (i, 0)),
            pl.BlockSpec((block_tokens, TOP_K), lambda i: (i, 0)),
            pl.BlockSpec((ER, h), lambda i: (0, 0)),
            pl.BlockSpec((ER, ff), lambda i: (0, 0)),
            pl.BlockSpec((ER, ff), lambda i: (0, 0)),
            pl.BlockSpec((ER, h), lambda i: (0, 0)),
        ],
        out_specs=pl.BlockSpec((block_tokens, 1, h), lambda i: (i, 0, 0)),
        out_shape=jax.ShapeDtypeStruct((n, b, h), jnp.float32),
        compiler_params=pltpu.CompilerParams(
            dimension_semantics=("parallel",)),
    )(x, ew, ei, w1_A, w1_B, w2_A, w2_B)


def kernel(x, expert_weights, expert_indices, w1_A, w1_B, w2_A, w2_B):
    ei = expert_indices.astype(jnp.int32)
    return _moe(x, expert_weights, ei, w1_A, w1_B, w2_A, w2_B,
                block_tokens=256)
